# baseline (speedup 1.0000x reference)
import jax
import jax.numpy as jnp
from jax.experimental import pallas as pl
from jax.experimental.pallas import tpu as pltpu

_N = 10000
_C = 128
_ROWS = 50
_BZ = 2000


def _body(x_ref, o_ref, zbuf, dsem, zsem):
    zbuf[...] = jnp.zeros(zbuf.shape, zbuf.dtype)
    copies = []
    for b in range(_N // _BZ):
        base = b * _BZ
        copies.append(
            pltpu.make_async_copy(
                zbuf, o_ref.at[pl.ds(base, _BZ), pl.ds(1, 24), :], zsem
            )
        )
        copies.append(
            pltpu.make_async_copy(
                zbuf, o_ref.at[pl.ds(base, _BZ), pl.ds(26, 24), :], zsem
            )
        )
    copies.append(
        pltpu.make_async_copy(
            x_ref.at[:, pl.ds(0, 1), :], o_ref.at[:, pl.ds(0, 1), :], dsem
        )
    )
    copies.append(
        pltpu.make_async_copy(
            x_ref.at[:, pl.ds(1, 1), :], o_ref.at[:, pl.ds(25, 1), :], dsem
        )
    )
    for c in copies:
        c.start()
    for c in copies:
        c.wait()


def kernel(atom_embeddings):
    x3 = atom_embeddings.reshape(_N, 2, _C)
    return pl.pallas_call(
        _body,
        in_specs=[pl.BlockSpec(memory_space=pltpu.MemorySpace.HBM)],
        out_specs=pl.BlockSpec(memory_space=pltpu.MemorySpace.HBM),
        out_shape=jax.ShapeDtypeStruct((_N, _ROWS, _C), x3.dtype),
        scratch_shapes=[
            pltpu.VMEM((_BZ, 24, _C), jnp.float32),
            pltpu.SemaphoreType.DMA,
            pltpu.SemaphoreType.DMA,
        ],
    )(x3)


# speedup vs baseline: 1.9807x; 1.9807x over previous
import jax
import jax.numpy as jnp
from jax.experimental import pallas as pl
from jax.experimental.pallas import tpu as pltpu

_N = 10000
_C = 128
_ROWS = 50
_A = 400
_NBLK = _N // _A
_NBUF = 3


def _body(x_hbm, o_hbm, *refs):
    bufs = refs[:_NBUF]
    xvs = refs[_NBUF:2 * _NBUF]
    isem, osem = refs[2 * _NBUF], refs[2 * _NBUF + 1]

    def in_copy(blk, b):
        return pltpu.make_async_copy(
            x_hbm.at[pl.ds(blk * _A, _A), :], xvs[b], isem.at[b]
        )

    def out_copy(blk, b):
        return pltpu.make_async_copy(
            bufs[b], o_hbm.at[pl.ds(blk * _A, _A), :, :], osem.at[b]
        )

    for b in range(_NBUF):
        in_copy(b, b).start()

    for i in range(_NBLK):
        b = i % _NBUF
        if i >= _NBUF:
            out_copy(i - _NBUF, b).wait()
        else:
            bufs[b][...] = jnp.zeros(bufs[b].shape, bufs[b].dtype)
        in_copy(i, b).wait()
        xb = xvs[b][...]
        bufs[b][:, 0:1, :] = xb[:, :_C].reshape(_A, 1, _C)
        bufs[b][:, 25:26, :] = xb[:, _C:].reshape(_A, 1, _C)
        out_copy(i, b).start()
        if i + _NBUF < _NBLK:
            in_copy(i + _NBUF, b).start()

    for i in range(_NBLK - _NBUF, _NBLK):
        out_copy(i, i % _NBUF).wait()


def kernel(atom_embeddings):
    return pl.pallas_call(
        _body,
        in_specs=[pl.BlockSpec(memory_space=pltpu.MemorySpace.HBM)],
        out_specs=pl.BlockSpec(memory_space=pltpu.MemorySpace.HBM),
        out_shape=jax.ShapeDtypeStruct((_N, _ROWS, _C), atom_embeddings.dtype),
        scratch_shapes=[pltpu.VMEM((_A, _ROWS, _C), jnp.float32)] * _NBUF
        + [pltpu.VMEM((_A, 2 * _C), jnp.float32)] * _NBUF
        + [
            pltpu.SemaphoreType.DMA((_NBUF,)),
            pltpu.SemaphoreType.DMA((_NBUF,)),
        ],
    )(atom_embeddings)


# speedup vs baseline: 1.9846x; 1.0020x over previous
import jax
import jax.numpy as jnp
from jax.experimental import pallas as pl
from jax.experimental.pallas import tpu as pltpu

_N = 10000
_C = 128
_ROWS = 50
_A = 400
_NBLK = _N // _A
_NBUF = 3


def _body(x_hbm, o_hbm, *refs):
    bufs = refs[:_NBUF]
    xvs = refs[_NBUF:2 * _NBUF]
    isem, osem = refs[2 * _NBUF], refs[2 * _NBUF + 1]

    def in_copy(blk, b):
        return pltpu.make_async_copy(
            x_hbm.at[pl.ds(blk * _A, _A), :], xvs[b], isem.at[b]
        )

    def out_copy(blk, b):
        return pltpu.make_async_copy(
            bufs[b], o_hbm.at[pl.ds(blk * _A, _A), :, :], osem.at[b]
        )

    for b in range(_NBUF):
        in_copy(b, b).start()

    for i in range(_NBLK):
        b = i % _NBUF
        if i >= _NBUF:
            out_copy(i - _NBUF, b).wait()
        else:
            bufs[b][...] = jnp.zeros(bufs[b].shape, bufs[b].dtype)
        in_copy(i, b).wait()
        xb = xvs[b][...]
        bufs[b][:, 0:1, :] = xb[:, :_C].reshape(_A, 1, _C)
        bufs[b][:, 25:26, :] = xb[:, _C:].reshape(_A, 1, _C)
        out_copy(i, b).start(priority=i % 2)
        if i + _NBUF < _NBLK:
            in_copy(i + _NBUF, b).start()

    for i in range(_NBLK - _NBUF, _NBLK):
        out_copy(i, i % _NBUF).wait()


def kernel(atom_embeddings):
    return pl.pallas_call(
        _body,
        in_specs=[pl.BlockSpec(memory_space=pltpu.MemorySpace.HBM)],
        out_specs=pl.BlockSpec(memory_space=pltpu.MemorySpace.HBM),
        out_shape=jax.ShapeDtypeStruct((_N, _ROWS, _C), atom_embeddings.dtype),
        scratch_shapes=[pltpu.VMEM((_A, _ROWS, _C), jnp.float32)] * _NBUF
        + [pltpu.VMEM((_A, 2 * _C), jnp.float32)] * _NBUF
        + [
            pltpu.SemaphoreType.DMA((_NBUF,)),
            pltpu.SemaphoreType.DMA((_NBUF,)),
        ],
    )(atom_embeddings)
